# Initial kernel scaffold; baseline (speedup 1.0000x reference)
import jax
import jax.numpy as jnp
from jax.experimental import pallas as pl


def kernel(t, control_points, joint_points):
    raise NotImplementedError("write your pallas kernel here")


# speedup vs baseline: 13.9723x; 13.9723x over previous
import functools

import jax
import jax.numpy as jnp
from jax import lax
from jax.experimental import pallas as pl
from jax.experimental.pallas import tpu as pltpu
from jax.experimental.pallas import tpu_sc as plsc

NUM_CORES = 2
NUM_SUBCORES = 16
NW = NUM_CORES * NUM_SUBCORES
LANES = 16

BLK = 4096


def _spline_sc(t, jp0_t, jp1_t, cp0_t, cp1_t, n_t, n_iv):
    n_per_w = n_t // NW
    n_blk = n_per_w // BLK
    j_iter = BLK // LANES
    build_chunks = n_iv // LANES

    mesh = plsc.VectorSubcoreMesh(
        core_axis_name="c", subcore_axis_name="s",
        num_cores=NUM_CORES, num_subcores=NUM_SUBCORES)

    @functools.partial(
        pl.kernel,
        out_type=jax.ShapeDtypeStruct((2 * n_t * 3,), jnp.float32),
        mesh=mesh,
        compiler_params=pltpu.CompilerParams(needs_layout_passes=False),
        scratch_types=[
            pltpu.VMEM(jp0_t.shape, jnp.float32),
            pltpu.VMEM(jp1_t.shape, jnp.float32),
            pltpu.VMEM(cp0_t.shape, jnp.float32),
            pltpu.VMEM(cp1_t.shape, jnp.float32),
            pltpu.VMEM((24 * n_iv,), jnp.float32),
            pltpu.VMEM((BLK,), jnp.float32),
            pltpu.VMEM((2 * BLK * 3,), jnp.float32),
        ],
    )
    def body(t_hbm, jp0_hbm, jp1_hbm, cp0_hbm, cp1_hbm, out_hbm,
             jp0_v, jp1_v, cp0_v, cp1_v, coef_v, t_v, out_v):
        wid = lax.axis_index("s") * NUM_CORES + lax.axis_index("c")
        base = wid * n_per_w

        pltpu.sync_copy(jp0_hbm, jp0_v)
        pltpu.sync_copy(jp1_hbm, jp1_v)
        pltpu.sync_copy(cp0_hbm, cp0_v)
        pltpu.sync_copy(cp1_hbm, cp1_v)

        def build_chunk(k, carry):
            s = k * LANES
            for b in range(2):
                for d in range(3):
                    col = b * 3 + d
                    p0 = jp0_v[b, d, pl.ds(s, LANES)]
                    p3 = jp1_v[b, d, pl.ds(s, LANES)]
                    p1 = cp0_v[b, d, pl.ds(s, LANES)]
                    p2 = cp1_v[b, d, pl.ds(s, LANES)]
                    c1 = 3.0 * (p1 - p0)
                    c2 = 3.0 * (p0 + p2) - 6.0 * p1
                    c3 = (p3 - p0) + 3.0 * (p1 - p2)
                    coef_v[pl.ds((0 * 6 + col) * n_iv + s, LANES)] = p0
                    coef_v[pl.ds((1 * 6 + col) * n_iv + s, LANES)] = c1
                    coef_v[pl.ds((2 * 6 + col) * n_iv + s, LANES)] = c2
                    coef_v[pl.ds((3 * 6 + col) * n_iv + s, LANES)] = c3
            return carry

        lax.fori_loop(0, build_chunks, build_chunk, 0)

        iota = lax.iota(jnp.int32, LANES)
        iota3 = iota * 3
        t_hi = jnp.float32(1.0 - 1e-10)
        n_iv_f = jnp.float32(n_iv)

        def jbody(j, carry):
            tt = t_v[pl.ds(j * LANES, LANES)]
            u = jnp.minimum(tt, t_hi) * n_iv_f
            idxr = u.astype(jnp.int32)
            uf = idxr.astype(jnp.float32)
            last = idxr >= n_iv
            idx = jnp.minimum(idxr, n_iv - 1)
            ptv = jnp.where(last, jnp.float32(1.0), u - uf)

            cs = [plsc.load_gather(coef_v, [idx + (r * n_iv)])
                  for r in range(24)]

            n3 = iota3 + j * (LANES * 3)
            for b in range(2):
                for d in range(3):
                    col = b * 3 + d
                    o = cs[18 + col]
                    o = o * ptv + cs[12 + col]
                    o = o * ptv + cs[6 + col]
                    o = o * ptv + cs[col]
                    plsc.store_scatter(out_v, [n3 + (b * (BLK * 3) + d)], o)
            return carry

        def do_block(blk, carry):
            n0 = base + blk * BLK
            pltpu.sync_copy(t_hbm.at[pl.ds(n0, BLK)], t_v)
            lax.fori_loop(0, j_iter, jbody, 0)
            pltpu.sync_copy(out_v.at[pl.ds(0, BLK * 3)],
                            out_hbm.at[pl.ds(n0 * 3, BLK * 3)])
            pltpu.sync_copy(out_v.at[pl.ds(BLK * 3, BLK * 3)],
                            out_hbm.at[pl.ds((n_t + n0) * 3, BLK * 3)])
            return carry

        lax.fori_loop(0, n_blk, do_block, 0)

    return body(t, jp0_t, jp1_t, cp0_t, cp1_t)


def kernel(t, control_points, joint_points):
    n_t = t.shape[0]
    batch, m_joints, dim = joint_points.shape
    n_iv = m_joints - 1

    jp0_t = jnp.transpose(joint_points[:, :n_iv, :], (0, 2, 1))
    jp1_t = jnp.transpose(joint_points[:, 1:, :], (0, 2, 1))
    cpr = control_points.reshape(batch, n_iv, 2, dim)
    cp0_t = jnp.transpose(cpr[:, :, 0, :], (0, 2, 1))
    cp1_t = jnp.transpose(cpr[:, :, 1, :], (0, 2, 1))

    flat = _spline_sc(t, jp0_t, jp1_t, cp0_t, cp1_t, n_t, n_iv)
    return flat.reshape(batch, n_t, dim)


# speedup vs baseline: 187.2618x; 13.4024x over previous
import functools

import jax
import jax.numpy as jnp
from jax import lax
from jax.experimental import pallas as pl
from jax.experimental.pallas import tpu as pltpu
from jax.experimental.pallas import tpu_sc as plsc

NUM_CORES = 2
NUM_SUBCORES = 16
NW = NUM_CORES * NUM_SUBCORES
LANES = 16

BLK = 4096


def _spline_sc(t, cp_flat, jp_flat, n_t, n_iv):
    n_per_w = n_t // NW
    n_blk = n_per_w // BLK
    j_iter = BLK // LANES
    build_chunks = n_iv // LANES
    jp_stride = (n_iv + 1) * 3
    cp_stride = n_iv * 2 * 3

    mesh = plsc.VectorSubcoreMesh(
        core_axis_name="c", subcore_axis_name="s",
        num_cores=NUM_CORES, num_subcores=NUM_SUBCORES)

    @functools.partial(
        pl.kernel,
        out_type=jax.ShapeDtypeStruct((3, n_t // 128, 2, 128), jnp.float32),
        mesh=mesh,
        compiler_params=pltpu.CompilerParams(needs_layout_passes=False),
        scratch_types=[
            pltpu.VMEM(jp_flat.shape, jnp.float32),
            pltpu.VMEM(cp_flat.shape, jnp.float32),
            pltpu.VMEM((24 * n_iv,), jnp.float32),
            pltpu.VMEM((BLK,), jnp.float32),
            pltpu.VMEM((3, BLK // 128, 2, 128), jnp.float32),
        ],
    )
    def body(t_hbm, cp_hbm, jp_hbm, out_hbm,
             jp_v, cp_v, coef_v, t_v, out_v):
        wid = lax.axis_index("s") * NUM_CORES + lax.axis_index("c")
        base = wid * n_per_w

        pltpu.sync_copy(jp_hbm, jp_v)
        pltpu.sync_copy(cp_hbm, cp_v)

        iota = lax.iota(jnp.int32, LANES)

        def build_chunk(k, carry):
            i_vec = k * LANES + iota
            i3 = i_vec * 3
            i6 = i_vec * 6
            for b in range(2):
                for d in range(3):
                    col = b * 3 + d
                    jo = b * jp_stride + d
                    co = b * cp_stride + d
                    p0 = plsc.load_gather(jp_v, [i3 + jo])
                    p3 = plsc.load_gather(jp_v, [i3 + (jo + 3)])
                    p1 = plsc.load_gather(cp_v, [i6 + co])
                    p2 = plsc.load_gather(cp_v, [i6 + (co + 3)])
                    c1 = 3.0 * (p1 - p0)
                    c2 = 3.0 * (p0 + p2) - 6.0 * p1
                    c3 = (p3 - p0) + 3.0 * (p1 - p2)
                    s = k * LANES
                    coef_v[pl.ds((0 * 6 + col) * n_iv + s, LANES)] = p0
                    coef_v[pl.ds((1 * 6 + col) * n_iv + s, LANES)] = c1
                    coef_v[pl.ds((2 * 6 + col) * n_iv + s, LANES)] = c2
                    coef_v[pl.ds((3 * 6 + col) * n_iv + s, LANES)] = c3
            return carry

        lax.fori_loop(0, build_chunks, build_chunk, 0)

        t_hi = jnp.float32(1.0 - 1e-10)
        n_iv_f = jnp.float32(n_iv)

        def jbody(j, carry):
            cur = carry[0]
            tt = t_v[pl.ds(j * LANES, LANES)]
            u = jnp.minimum(tt, t_hi) * n_iv_f
            idxr = u.astype(jnp.int32)
            uf = idxr.astype(jnp.float32)
            last = idxr >= n_iv
            idx = jnp.minimum(idxr, n_iv - 1)
            ptv = jnp.where(last, jnp.float32(1.0), u - uf)

            def fast():
                return carry + carry[1:25]

            def slow():
                new_cur = jnp.broadcast_to(jnp.max(idx), (LANES,))
                bcast = tuple(plsc.load_gather(coef_v, [new_cur + (r * n_iv)])
                              for r in range(24))
                lane = tuple(plsc.load_gather(coef_v, [idx + (r * n_iv)])
                             for r in range(24))
                return (new_cur,) + bcast + lane

            same = jnp.all(idx == cur)
            res = lax.cond(same, fast, slow)
            nxt = tuple(res[:25])
            cs = res[25:]

            kc = j // 8
            lane0 = (j % 8) * LANES
            for b in range(2):
                for d in range(3):
                    col = b * 3 + d
                    o = cs[18 + col]
                    o = o * ptv + cs[12 + col]
                    o = o * ptv + cs[6 + col]
                    o = o * ptv + cs[col]
                    out_v[d, kc, b, pl.ds(lane0, LANES)] = o
            return nxt

        def do_block(blk, carry):
            n0 = base + blk * BLK
            k0 = n0 // 128
            pltpu.sync_copy(t_hbm.at[pl.ds(n0, BLK)], t_v)
            carry = lax.fori_loop(0, j_iter, jbody, carry)
            for d in range(3):
                pltpu.sync_copy(
                    out_v.at[pl.ds(d, 1)],
                    out_hbm.at[pl.ds(d, 1), pl.ds(k0, BLK // 128)])
            return carry

        zero_v = jnp.zeros((LANES,), jnp.float32)
        init = (jnp.full((LANES,), -1, jnp.int32),) + (zero_v,) * 24
        lax.fori_loop(0, n_blk, do_block, init)

    return body(t, cp_flat, jp_flat)


def kernel(t, control_points, joint_points):
    n_t = t.shape[0]
    n_iv = joint_points.shape[1] - 1
    jp_flat = joint_points.reshape(-1)
    jp_flat = jnp.pad(jp_flat, (0, (-jp_flat.shape[0]) % 16))
    cp_flat = control_points.reshape(-1)
    out4 = _spline_sc(t, cp_flat, jp_flat, n_t, n_iv)
    return out4.transpose(2, 1, 3, 0).reshape(2, n_t, 3)


# speedup vs baseline: 353.3767x; 1.8871x over previous
import functools

import jax
import jax.numpy as jnp
from jax import lax
from jax.experimental import pallas as pl
from jax.experimental.pallas import tpu as pltpu
from jax.experimental.pallas import tpu_sc as plsc

NUM_CORES = 2
NUM_SUBCORES = 16
NW = NUM_CORES * NUM_SUBCORES
LANES = 16

BLK = 4096
GRP = 128


def _spline_sc(t, cp_flat, jp_flat, n_t, n_iv):
    n_per_w = n_t // NW
    n_blk = n_per_w // BLK
    g_iter = BLK // GRP
    vpg = GRP // LANES
    build_chunks = n_iv // LANES
    jp_stride = (n_iv + 1) * 3
    cp_stride = n_iv * 2 * 3

    mesh = plsc.VectorSubcoreMesh(
        core_axis_name="c", subcore_axis_name="s",
        num_cores=NUM_CORES, num_subcores=NUM_SUBCORES)

    @functools.partial(
        pl.kernel,
        out_type=jax.ShapeDtypeStruct((3, n_t // 128, 2, 128), jnp.float32),
        mesh=mesh,
        compiler_params=pltpu.CompilerParams(needs_layout_passes=False),
        scratch_types=[
            pltpu.VMEM(jp_flat.shape, jnp.float32),
            pltpu.VMEM(cp_flat.shape, jnp.float32),
            pltpu.VMEM((24 * n_iv,), jnp.float32),
            pltpu.VMEM((BLK,), jnp.float32),
            pltpu.VMEM((3, BLK // 128, 2, 128), jnp.float32),
        ],
    )
    def body(t_hbm, cp_hbm, jp_hbm, out_hbm,
             jp_v, cp_v, coef_v, t_v, out_v):
        wid = lax.axis_index("s") * NUM_CORES + lax.axis_index("c")
        base = wid * n_per_w

        pltpu.sync_copy(jp_hbm, jp_v)
        pltpu.sync_copy(cp_hbm, cp_v)

        iota = lax.iota(jnp.int32, LANES)

        def build_chunk(k, carry):
            i_vec = k * LANES + iota
            i3 = i_vec * 3
            i6 = i_vec * 6
            for b in range(2):
                for d in range(3):
                    col = b * 3 + d
                    jo = b * jp_stride + d
                    co = b * cp_stride + d
                    p0 = plsc.load_gather(jp_v, [i3 + jo])
                    p3 = plsc.load_gather(jp_v, [i3 + (jo + 3)])
                    p1 = plsc.load_gather(cp_v, [i6 + co])
                    p2 = plsc.load_gather(cp_v, [i6 + (co + 3)])
                    c1 = 3.0 * (p1 - p0)
                    c2 = 3.0 * (p0 + p2) - 6.0 * p1
                    c3 = (p3 - p0) + 3.0 * (p1 - p2)
                    s = k * LANES
                    coef_v[pl.ds((0 * 6 + col) * n_iv + s, LANES)] = p0
                    coef_v[pl.ds((1 * 6 + col) * n_iv + s, LANES)] = c1
                    coef_v[pl.ds((2 * 6 + col) * n_iv + s, LANES)] = c2
                    coef_v[pl.ds((3 * 6 + col) * n_iv + s, LANES)] = c3
            return carry

        lax.fori_loop(0, build_chunks, build_chunk, 0)

        t_hi = jnp.float32(1.0 - 1e-10)
        n_iv_f = jnp.float32(n_iv)

        def horner_store(g, v, ptv, cs):
            kc = g
            lane0 = v * LANES
            for b in range(2):
                for d in range(3):
                    col = b * 3 + d
                    o = cs[18 + col]
                    o = o * ptv + cs[12 + col]
                    o = o * ptv + cs[6 + col]
                    o = o * ptv + cs[col]
                    out_v[d, kc, b, pl.ds(lane0, LANES)] = o

        def gbody(g, carry):
            thr = carry[0]
            bcast = carry[1:]
            tv_last = t_v[pl.ds(g * GRP + (GRP - LANES), LANES)]
            u_last = tv_last[LANES - 1] * n_iv_f

            def fast():
                ptvs = []
                for v in range(vpg):
                    tt = t_v[pl.ds(g * GRP + v * LANES, LANES)]
                    u = tt * n_iv_f
                    uf = u.astype(jnp.int32).astype(jnp.float32)
                    ptvs.append(u - uf)
                for v in range(vpg):
                    horner_store(g, v, ptvs[v], bcast)
                return carry

            def slow():
                idx_last = jnp.int32(0)
                for v in range(vpg):
                    tt = t_v[pl.ds(g * GRP + v * LANES, LANES)]
                    u = jnp.minimum(tt, t_hi) * n_iv_f
                    idxr = u.astype(jnp.int32)
                    uf = idxr.astype(jnp.float32)
                    idx = jnp.minimum(idxr, n_iv - 1)
                    ptv = jnp.where(idxr >= n_iv, jnp.float32(1.0), u - uf)
                    cs = [plsc.load_gather(coef_v, [idx + (r * n_iv)])
                          for r in range(24)]
                    horner_store(g, v, ptv, cs)
                    if v == vpg - 1:
                        idx_last = jnp.max(idx)
                new_cur = jnp.broadcast_to(idx_last, (LANES,))
                nb = tuple(plsc.load_gather(coef_v, [new_cur + (r * n_iv)])
                           for r in range(24))
                new_thr = (idx_last + 1).astype(jnp.float32)
                return (new_thr,) + nb

            return lax.cond(u_last < thr, fast, slow)

        def do_block(blk, carry):
            n0 = base + blk * BLK
            k0 = n0 // 128
            pltpu.sync_copy(t_hbm.at[pl.ds(n0, BLK)], t_v)
            carry = lax.fori_loop(0, g_iter, gbody, carry)
            for d in range(3):
                pltpu.sync_copy(
                    out_v.at[pl.ds(d, 1)],
                    out_hbm.at[pl.ds(d, 1), pl.ds(k0, BLK // 128)])
            return carry

        zero_v = jnp.zeros((LANES,), jnp.float32)
        init = (jnp.float32(-1.0),) + (zero_v,) * 24
        lax.fori_loop(0, n_blk, do_block, init)

    return body(t, cp_flat, jp_flat)


def kernel(t, control_points, joint_points):
    n_t = t.shape[0]
    n_iv = joint_points.shape[1] - 1
    jp_flat = joint_points.reshape(-1)
    jp_flat = jnp.pad(jp_flat, (0, (-jp_flat.shape[0]) % 16))
    cp_flat = control_points.reshape(-1)
    out4 = _spline_sc(t, cp_flat, jp_flat, n_t, n_iv)
    return out4.transpose(2, 1, 3, 0).reshape(2, n_t, 3)


# speedup vs baseline: 481.1234x; 1.3615x over previous
import functools

import jax
import jax.numpy as jnp
from jax import lax
from jax.experimental import pallas as pl
from jax.experimental.pallas import tpu as pltpu
from jax.experimental.pallas import tpu_sc as plsc

NUM_CORES = 2
NUM_SUBCORES = 16
NW = NUM_CORES * NUM_SUBCORES
LANES = 16

BLK = 4096
GRP = 128


def _spline_sc(t, cp_flat, jp_flat, n_t, n_iv):
    n_per_w = n_t // NW
    n_blk = n_per_w // BLK
    g_iter = BLK // GRP
    vpg = GRP // LANES
    build_chunks = n_iv // LANES
    jp_stride = (n_iv + 1) * 3
    cp_stride = n_iv * 2 * 3

    mesh = plsc.VectorSubcoreMesh(
        core_axis_name="c", subcore_axis_name="s",
        num_cores=NUM_CORES, num_subcores=NUM_SUBCORES)

    @functools.partial(
        pl.kernel,
        out_type=jax.ShapeDtypeStruct((3, n_t // 128, 2, 128), jnp.float32),
        mesh=mesh,
        compiler_params=pltpu.CompilerParams(needs_layout_passes=False),
        scratch_types=[
            pltpu.VMEM(jp_flat.shape, jnp.float32),
            pltpu.VMEM(cp_flat.shape, jnp.float32),
            pltpu.VMEM((24 * n_iv,), jnp.float32),
            pltpu.VMEM((BLK,), jnp.float32),
            pltpu.VMEM((BLK,), jnp.float32),
            pltpu.VMEM((3, BLK // 128, 2, 128), jnp.float32),
            pltpu.VMEM((3, BLK // 128, 2, 128), jnp.float32),
            pltpu.SemaphoreType.DMA,
            pltpu.SemaphoreType.DMA,
            pltpu.SemaphoreType.DMA,
            pltpu.SemaphoreType.DMA,
        ],
    )
    def body(t_hbm, cp_hbm, jp_hbm, out_hbm,
             jp_v, cp_v, coef_v, t_va, t_vb, out_va, out_vb,
             st0, st1, so0, so1):
        wid = lax.axis_index("s") * NUM_CORES + lax.axis_index("c")
        base = wid * n_per_w

        pltpu.sync_copy(jp_hbm, jp_v)
        pltpu.sync_copy(cp_hbm, cp_v)

        iota = lax.iota(jnp.int32, LANES)

        def build_chunk(k, carry):
            i_vec = k * LANES + iota
            i3 = i_vec * 3
            i6 = i_vec * 6
            for b in range(2):
                for d in range(3):
                    col = b * 3 + d
                    jo = b * jp_stride + d
                    co = b * cp_stride + d
                    p0 = plsc.load_gather(jp_v, [i3 + jo])
                    p3 = plsc.load_gather(jp_v, [i3 + (jo + 3)])
                    p1 = plsc.load_gather(cp_v, [i6 + co])
                    p2 = plsc.load_gather(cp_v, [i6 + (co + 3)])
                    c1 = 3.0 * (p1 - p0)
                    c2 = 3.0 * (p0 + p2) - 6.0 * p1
                    c3 = (p3 - p0) + 3.0 * (p1 - p2)
                    s = k * LANES
                    coef_v[pl.ds((0 * 6 + col) * n_iv + s, LANES)] = p0
                    coef_v[pl.ds((1 * 6 + col) * n_iv + s, LANES)] = c1
                    coef_v[pl.ds((2 * 6 + col) * n_iv + s, LANES)] = c2
                    coef_v[pl.ds((3 * 6 + col) * n_iv + s, LANES)] = c3
            return carry

        lax.fori_loop(0, build_chunks, build_chunk, 0)

        t_hi = jnp.float32(1.0 - 1e-10)
        n_iv_f = jnp.float32(n_iv)

        def horner_store(out_v, g, v, ptv, cs):
            kc = g
            lane0 = v * LANES
            for b in range(2):
                for d in range(3):
                    col = b * 3 + d
                    o = cs[18 + col]
                    o = o * ptv + cs[12 + col]
                    o = o * ptv + cs[6 + col]
                    o = o * ptv + cs[col]
                    out_v[d, kc, b, pl.ds(lane0, LANES)] = o

        def make_gbody(t_v, out_v):
            def gbody(g, carry):
                thr = carry[0]
                bcast = carry[1:]
                tv_last = t_v[pl.ds(g * GRP + (GRP - LANES), LANES)]
                u_last = tv_last[LANES - 1] * n_iv_f

                def fast():
                    ptvs = []
                    for v in range(vpg):
                        tt = t_v[pl.ds(g * GRP + v * LANES, LANES)]
                        u = tt * n_iv_f
                        uf = u.astype(jnp.int32).astype(jnp.float32)
                        ptvs.append(u - uf)
                    for v in range(vpg):
                        horner_store(out_v, g, v, ptvs[v], bcast)
                    return carry

                def slow():
                    idx_last = jnp.int32(0)
                    for v in range(vpg):
                        tt = t_v[pl.ds(g * GRP + v * LANES, LANES)]
                        u = jnp.minimum(tt, t_hi) * n_iv_f
                        idxr = u.astype(jnp.int32)
                        uf = idxr.astype(jnp.float32)
                        idx = jnp.minimum(idxr, n_iv - 1)
                        ptv = jnp.where(idxr >= n_iv, jnp.float32(1.0),
                                        u - uf)
                        cs = [plsc.load_gather(coef_v, [idx + (r * n_iv)])
                              for r in range(24)]
                        horner_store(out_v, g, v, ptv, cs)
                        if v == vpg - 1:
                            idx_last = jnp.max(idx)
                    new_cur = jnp.broadcast_to(idx_last, (LANES,))
                    nb = tuple(plsc.load_gather(coef_v,
                                                [new_cur + (r * n_iv)])
                               for r in range(24))
                    new_thr = (idx_last + 1).astype(jnp.float32)
                    return (new_thr,) + nb

                return lax.cond(u_last < thr, fast, slow)
            return gbody

        def t_slice(blk):
            return t_hbm.at[pl.ds(base + blk * BLK, BLK)]

        def out_pairs(blk, out_v):
            k0 = (base + blk * BLK) // 128
            return [(out_v.at[pl.ds(d, 1)],
                     out_hbm.at[pl.ds(d, 1), pl.ds(k0, BLK // 128)])
                    for d in range(3)]

        bufs = ((t_va, out_va, st0, so0), (t_vb, out_vb, st1, so1))
        pltpu.async_copy(t_slice(0), t_va, st0)

        def pair_body(p, carry):
            for h in range(2):
                blk = p * 2 + h
                t_v, out_v, st_, so_ = bufs[h]
                nt_v, _, nst_, _ = bufs[1 - h]

                if h == 0:
                    pltpu.async_copy(t_slice(blk + 1), nt_v, nst_)
                else:
                    @pl.when(blk + 1 < n_blk)
                    def _():
                        pltpu.async_copy(t_slice(blk + 1), nt_v, nst_)

                pltpu.make_async_copy(t_slice(blk), t_v, st_).wait()

                @pl.when(p > 0)
                def _():
                    for s_, d_ in out_pairs(blk - 2, out_v):
                        pltpu.make_async_copy(s_, d_, so_).wait()

                carry = lax.fori_loop(0, g_iter, make_gbody(t_v, out_v),
                                      carry)
                for s_, d_ in out_pairs(blk, out_v):
                    pltpu.async_copy(s_, d_, so_)
            return carry

        zero_v = jnp.zeros((LANES,), jnp.float32)
        init = (jnp.float32(-1.0),) + (zero_v,) * 24
        lax.fori_loop(0, n_blk // 2, pair_body, init)
        for s_, d_ in out_pairs(n_blk - 2, out_va):
            pltpu.make_async_copy(s_, d_, so0).wait()
        for s_, d_ in out_pairs(n_blk - 1, out_vb):
            pltpu.make_async_copy(s_, d_, so1).wait()

    return body(t, cp_flat, jp_flat)


def kernel(t, control_points, joint_points):
    n_t = t.shape[0]
    n_iv = joint_points.shape[1] - 1
    jp_flat = joint_points.reshape(-1)
    jp_flat = jnp.pad(jp_flat, (0, (-jp_flat.shape[0]) % 16))
    cp_flat = control_points.reshape(-1)
    out4 = _spline_sc(t, cp_flat, jp_flat, n_t, n_iv)
    return out4.transpose(2, 1, 3, 0).reshape(2, n_t, 3)


# speedup vs baseline: 500.7549x; 1.0408x over previous
import functools

import jax
import jax.numpy as jnp
from jax import lax
from jax.experimental import pallas as pl
from jax.experimental.pallas import tpu as pltpu
from jax.experimental.pallas import tpu_sc as plsc

NUM_CORES = 2
NUM_SUBCORES = 16
NW = NUM_CORES * NUM_SUBCORES
LANES = 16

BLK = 4096
GRP = 128


def _spline_sc(t, cp_flat, jp_flat, n_t, n_iv):
    n_per_w = n_t // NW
    n_blk = n_per_w // BLK
    g_iter = BLK // GRP
    vpg = GRP // LANES
    build_chunks = n_iv // LANES
    jp_stride = (n_iv + 1) * 3
    cp_stride = n_iv * 2 * 3

    mesh = plsc.VectorSubcoreMesh(
        core_axis_name="c", subcore_axis_name="s",
        num_cores=NUM_CORES, num_subcores=NUM_SUBCORES)

    @functools.partial(
        pl.kernel,
        out_type=jax.ShapeDtypeStruct((3, n_t // 128, 2, 128), jnp.float32),
        mesh=mesh,
        compiler_params=pltpu.CompilerParams(needs_layout_passes=False),
        scratch_types=[
            pltpu.VMEM(jp_flat.shape, jnp.float32),
            pltpu.VMEM(cp_flat.shape, jnp.float32),
            pltpu.VMEM((24 * n_iv,), jnp.float32),
            pltpu.VMEM((BLK,), jnp.float32),
            pltpu.VMEM((BLK,), jnp.float32),
            pltpu.VMEM((3, BLK // 128, 2, 128), jnp.float32),
            pltpu.VMEM((3, BLK // 128, 2, 128), jnp.float32),
            pltpu.SemaphoreType.DMA,
            pltpu.SemaphoreType.DMA,
            pltpu.SemaphoreType.DMA,
            pltpu.SemaphoreType.DMA,
        ],
    )
    def body(t_hbm, cp_hbm, jp_hbm, out_hbm,
             jp_v, cp_v, coef_v, t_va, t_vb, out_va, out_vb,
             st0, st1, so0, so1):
        wid = lax.axis_index("s") * NUM_CORES + lax.axis_index("c")
        base = wid * n_per_w

        pltpu.async_copy(t_hbm.at[pl.ds(base, BLK)], t_va, st0)

        pltpu.sync_copy(jp_hbm, jp_v)
        pltpu.sync_copy(cp_hbm, cp_v)

        iota = lax.iota(jnp.int32, LANES)

        def build_chunk(k, carry):
            i_vec = k * LANES + iota
            i3 = i_vec * 3
            i6 = i_vec * 6
            for b in range(2):
                for d in range(3):
                    col = b * 3 + d
                    jo = b * jp_stride + d
                    co = b * cp_stride + d
                    p0 = plsc.load_gather(jp_v, [i3 + jo])
                    p3 = plsc.load_gather(jp_v, [i3 + (jo + 3)])
                    p1 = plsc.load_gather(cp_v, [i6 + co])
                    p2 = plsc.load_gather(cp_v, [i6 + (co + 3)])
                    c1 = 3.0 * (p1 - p0)
                    c2 = 3.0 * (p0 + p2) - 6.0 * p1
                    c3 = (p3 - p0) + 3.0 * (p1 - p2)
                    s = k * LANES
                    coef_v[pl.ds((0 * 6 + col) * n_iv + s, LANES)] = p0
                    coef_v[pl.ds((1 * 6 + col) * n_iv + s, LANES)] = c1
                    coef_v[pl.ds((2 * 6 + col) * n_iv + s, LANES)] = c2
                    coef_v[pl.ds((3 * 6 + col) * n_iv + s, LANES)] = c3
            return carry

        lax.fori_loop(0, build_chunks, build_chunk, 0)

        t_hi = jnp.float32(1.0 - 1e-10)
        n_iv_f = jnp.float32(n_iv)

        def horner_store(out_v, g, v, ptv, cs):
            kc = g
            lane0 = v * LANES
            for b in range(2):
                for d in range(3):
                    col = b * 3 + d
                    o = cs[18 + col]
                    o = o * ptv + cs[12 + col]
                    o = o * ptv + cs[6 + col]
                    o = o * ptv + cs[col]
                    out_v[d, kc, b, pl.ds(lane0, LANES)] = o

        def last_vec(t_v, g):
            return t_v[pl.ds(g * GRP + (GRP - LANES), LANES)]

        def make_gbody(t_v, out_v):
            def gbody(g, carry):
                u_chk = carry[0]
                tnext = carry[1]
                thr = carry[2]
                bcast = carry[3:]
                g2 = jnp.minimum(g + 2, g_iter - 1)

                def stage():
                    u_next = tnext[LANES - 1] * n_iv_f
                    return u_next, last_vec(t_v, g2)

                def fast():
                    u_next, t2 = stage()
                    ptvs = []
                    for v in range(vpg):
                        tt = t_v[pl.ds(g * GRP + v * LANES, LANES)]
                        u = tt * n_iv_f
                        uf = u.astype(jnp.int32).astype(jnp.float32)
                        ptvs.append(u - uf)
                    for v in range(vpg):
                        horner_store(out_v, g, v, ptvs[v], bcast)
                    return (u_next, t2, thr) + bcast

                def slow():
                    u_next, t2 = stage()
                    idx_last = jnp.int32(0)
                    for v in range(vpg):
                        tt = t_v[pl.ds(g * GRP + v * LANES, LANES)]
                        u = jnp.minimum(tt, t_hi) * n_iv_f
                        idxr = u.astype(jnp.int32)
                        uf = idxr.astype(jnp.float32)
                        idx = jnp.minimum(idxr, n_iv - 1)
                        ptv = jnp.where(idxr >= n_iv, jnp.float32(1.0),
                                        u - uf)
                        cs = [plsc.load_gather(coef_v, [idx + (r * n_iv)])
                              for r in range(24)]
                        horner_store(out_v, g, v, ptv, cs)
                        if v == vpg - 1:
                            idx_last = jnp.max(idx)
                    new_cur = jnp.broadcast_to(idx_last, (LANES,))
                    nb = tuple(plsc.load_gather(coef_v,
                                                [new_cur + (r * n_iv)])
                               for r in range(24))
                    new_thr = (idx_last + 1).astype(jnp.float32)
                    return (u_next, t2, new_thr) + nb

                return lax.cond(u_chk < thr, fast, slow)
            return gbody

        def t_slice(blk):
            return t_hbm.at[pl.ds(base + blk * BLK, BLK)]

        def out_pairs(blk, out_v):
            k0 = (base + blk * BLK) // 128
            return [(out_v.at[pl.ds(d, 1)],
                     out_hbm.at[pl.ds(d, 1), pl.ds(k0, BLK // 128)])
                    for d in range(3)]

        bufs = ((t_va, out_va, st0, so0), (t_vb, out_vb, st1, so1))

        def pair_body(p, carry):
            for h in range(2):
                blk = p * 2 + h
                t_v, out_v, st_, so_ = bufs[h]
                nt_v, _, nst_, _ = bufs[1 - h]

                if h == 0:
                    pltpu.async_copy(t_slice(blk + 1), nt_v, nst_)
                else:
                    @pl.when(blk + 1 < n_blk)
                    def _():
                        pltpu.async_copy(t_slice(blk + 1), nt_v, nst_)

                pltpu.make_async_copy(t_slice(blk), t_v, st_).wait()

                @pl.when(p > 0)
                def _():
                    for s_, d_ in out_pairs(blk - 2, out_v):
                        pltpu.make_async_copy(s_, d_, so_).wait()

                u0 = last_vec(t_v, 0)[LANES - 1] * n_iv_f
                t1 = last_vec(t_v, 1)
                res = lax.fori_loop(0, g_iter, make_gbody(t_v, out_v),
                                    (u0, t1) + carry)
                carry = tuple(res[2:])
                for s_, d_ in out_pairs(blk, out_v):
                    pltpu.async_copy(s_, d_, so_)
            return carry

        zero_v = jnp.zeros((LANES,), jnp.float32)
        init = (jnp.float32(-1.0),) + (zero_v,) * 24
        lax.fori_loop(0, n_blk // 2, pair_body, init)
        for s_, d_ in out_pairs(n_blk - 2, out_va):
            pltpu.make_async_copy(s_, d_, so0).wait()
        for s_, d_ in out_pairs(n_blk - 1, out_vb):
            pltpu.make_async_copy(s_, d_, so1).wait()

    return body(t, cp_flat, jp_flat)


def kernel(t, control_points, joint_points):
    n_t = t.shape[0]
    n_iv = joint_points.shape[1] - 1
    jp_flat = joint_points.reshape(-1)
    jp_flat = jnp.pad(jp_flat, (0, (-jp_flat.shape[0]) % 16))
    cp_flat = control_points.reshape(-1)
    out4 = _spline_sc(t, cp_flat, jp_flat, n_t, n_iv)
    return out4.transpose(2, 1, 3, 0).reshape(2, n_t, 3)
